# baseline (speedup 1.0000x reference)
import functools

import jax
import jax.numpy as jnp
from jax import lax
from jax.experimental import pallas as pl
from jax.experimental.pallas import tpu as pltpu
from jax.experimental.pallas import tpu_sc as plsc

N = 10000
E = 320000
D = 128
G = 64

NC = 2
NS = 16
NW = NC * NS
CH = 128
NCHUNK = 80
EPT = NCHUNK * CH
E_PAD = EPT * NW
N_PAD = 10240
RPT = N_PAD // NS

_F32 = jnp.float32
_MESH = plsc.VectorSubcoreMesh(core_axis_name="c", subcore_axis_name="s")


WIN = 8
NWIN = NCHUNK // WIN
NCK0 = 160
NCK1 = 0
NCK_MAX = 160


def _zero_fill(buf, val):
    v16 = jnp.full((16,), val, _F32)

    def _row(r, carry):
        for q in range(D // 16):
            buf[r, pl.ds(q * 16, 16)] = v16
        return carry

    lax.fori_loop(0, CH, _row, 0)


def _zero_agg(buf, agg_sh, s):
    _zero_fill(buf, 0.0)
    r0 = s * RPT
    for k in range(RPT // CH):
        pltpu.sync_copy(buf, agg_sh.at[pl.ds(r0 + k * CH, CH)])


def _writeout(buf, agg_sh, out_hbm, c, s):
    r0 = s * RPT
    for k in range(RPT // CH):
        pltpu.sync_copy(agg_sh.at[pl.ds(r0 + k * CH, CH)], buf)
        pltpu.sync_copy(buf, out_hbm.at[c, pl.ds(r0 + k * CH, CH)])


@functools.partial(
    pl.kernel,
    out_type=jax.ShapeDtypeStruct((NC, N_PAD, D), _F32),
    mesh=_MESH,
    scratch_types=[
        pltpu.VMEM((WIN, CH), jnp.int32),
        pltpu.VMEM((WIN, CH), jnp.int32),
        pltpu.VMEM((CH, D), _F32),
        pltpu.VMEM((CH, D), _F32),
        pltpu.VMEM_SHARED((N_PAD, D), _F32),
        pltpu.SemaphoreType.DMA,
        pltpu.SemaphoreType.DMA,
    ],
)
def _sc_scatter_add(a_hbm, src_hbm, dst_hbm, out_hbm,
                    src_w, dst_w, buf0, buf1, agg_sh, sem0, sem1):
    c = lax.axis_index("c")
    s = lax.axis_index("s")
    wid = c * NS + s
    nwin = lax.select(c == 0, NCK0 // WIN, NCK1 // WIN)

    _zero_agg(buf0, agg_sh, s)
    plsc.subcore_barrier()

    bufs = (buf0, buf1)
    sems = (sem0, sem1)

    def _window(w, carry):
        pltpu.sync_copy(src_hbm.at[wid, pl.ds(w * WIN, WIN)], src_w)
        pltpu.sync_copy(dst_hbm.at[wid, pl.ds(w * WIN, WIN)], dst_w)
        pending = pltpu.async_copy(a_hbm.at[src_w.at[0]], buf0, sem0)
        for k in range(WIN):
            nxt = None
            if k + 1 < WIN:
                nxt = pltpu.async_copy(
                    a_hbm.at[src_w.at[k + 1]], bufs[(k + 1) % 2],
                    sems[(k + 1) % 2])
            pending.wait()
            pltpu.sync_copy(bufs[k % 2], agg_sh.at[dst_w.at[k]], add=True)
            pending = nxt
        return carry

    lax.fori_loop(0, nwin, _window, 0)
    plsc.subcore_barrier()
    _writeout(buf0, agg_sh, out_hbm, c, s)


@functools.partial(
    pl.kernel,
    out_type=jax.ShapeDtypeStruct((NC, N_PAD, D), _F32),
    mesh=_MESH,
    scratch_types=[
        pltpu.VMEM((WIN, CH), jnp.int32),
        pltpu.VMEM((CH, D), _F32),
        pltpu.VMEM_SHARED((N_PAD, D), _F32),
    ],
)
def _sc_degree(dst_hbm, out_hbm, dst_w, buf, agg_sh):
    c = lax.axis_index("c")
    s = lax.axis_index("s")
    wid = c * NS + s
    nwin = lax.select(c == 0, NCK0 // WIN, NCK1 // WIN)

    _zero_agg(buf, agg_sh, s)
    plsc.subcore_barrier()
    _zero_fill(buf, 1.0)

    def _window(w, carry):
        pltpu.sync_copy(dst_hbm.at[wid, pl.ds(w * WIN, WIN)], dst_w)
        for k in range(WIN):
            pltpu.sync_copy(buf, agg_sh.at[dst_w.at[k]], add=True)
        return carry

    lax.fori_loop(0, nwin, _window, 0)
    plsc.subcore_barrier()
    _writeout(buf, agg_sh, out_hbm, c, s)


BR = 2560
NBLK = N_PAD // BR


def _dot(a, b):
    return jnp.dot(a, b, precision=lax.Precision.HIGHEST,
                   preferred_element_type=_F32)


def _pre_body(x_ref, wlT_ref, wrT_ref, a_ref, b_ref):
    xv = x_ref[...]
    a_ref[...] = _dot(xv, wlT_ref[...])
    b_ref[...] = _dot(xv, wrT_ref[...])


_W_SPEC = pl.BlockSpec((D, D), lambda i: (0, 0))
_ROW_SPEC = pl.BlockSpec((BR, D), lambda i: (i, 0))

_tc_pre = pl.pallas_call(
    _pre_body,
    grid=(NBLK,),
    in_specs=[_ROW_SPEC, _W_SPEC, _W_SPEC],
    out_specs=(_ROW_SPEC, _ROW_SPEC),
    out_shape=(jax.ShapeDtypeStruct((N_PAD, D), _F32),
               jax.ShapeDtypeStruct((N_PAD, D), _F32)),
)


def _relu_layer(p_ref, cnt_ref, b_ref, bias_ref):
    agg = p_ref[0] + p_ref[1]
    cnt = cnt_ref[:, 0:1] + cnt_ref[:, 1:2]
    inv = 1.0 / jnp.maximum(cnt, 1.0)
    h = jnp.maximum(agg * inv + bias_ref[...] + b_ref[...], 0.0)
    row0 = pl.program_id(0) * BR
    mask = (row0 + lax.broadcasted_iota(jnp.int32, (BR, 1), 0) < N)
    return h * mask.astype(_F32)


_P_SPEC = pl.BlockSpec((NC, BR, D), lambda i: (0, i, 0))
_CNT_SPEC = pl.BlockSpec((BR, NC), lambda i: (i, 0))
_BIAS_SPEC = pl.BlockSpec((1, D), lambda i: (0, 0))


def _mid_body(p_ref, cnt_ref, b_ref, bias_ref, wlT_ref, wrT_ref,
              a_out, b_out):
    h = _relu_layer(p_ref, cnt_ref, b_ref, bias_ref)
    a_out[...] = _dot(h, wlT_ref[...])
    b_out[...] = _dot(h, wrT_ref[...])


_tc_mid = pl.pallas_call(
    _mid_body,
    grid=(NBLK,),
    in_specs=[_P_SPEC, _CNT_SPEC, _ROW_SPEC, _BIAS_SPEC, _W_SPEC, _W_SPEC],
    out_specs=(_ROW_SPEC, _ROW_SPEC),
    out_shape=(jax.ShapeDtypeStruct((N_PAD, D), _F32),
               jax.ShapeDtypeStruct((N_PAD, D), _F32)),
)


def _final_body(p_ref, cnt_ref, b_ref, bias_ref, batch_ref, out_ref):
    h = _relu_layer(p_ref, cnt_ref, b_ref, bias_ref)
    onehot = (lax.broadcasted_iota(jnp.int32, (G, BR), 0)
              == batch_ref[...]).astype(_F32)
    part = _dot(onehot, h)

    @pl.when(pl.program_id(0) == 0)
    def _init():
        out_ref[...] = jnp.zeros((G, D), _F32)

    out_ref[...] += part


_tc_final = pl.pallas_call(
    _final_body,
    grid=(NBLK,),
    in_specs=[_P_SPEC, _CNT_SPEC, _ROW_SPEC, _BIAS_SPEC,
              pl.BlockSpec((1, BR), lambda i: (0, i))],
    out_specs=pl.BlockSpec((G, D), lambda i: (0, 0)),
    out_shape=jax.ShapeDtypeStruct((G, D), _F32),
)


def kernel(x, edge_index, batch, Wl0, bl0, Wr0, Wl1, bl1, Wr1, Wl2, bl2, Wr2):
    src = edge_index[0].astype(jnp.int32)
    dst = edge_index[1].astype(jnp.int32)

    x_pad = jnp.zeros((N_PAD, D), _F32).at[:N].set(x)
    def _pack(e):
        flat = jnp.full((E_PAD,), N_PAD - 1, jnp.int32).at[:E].set(e)
        b0 = flat[:NS * NCK0 * CH].reshape(NS, NCK0, CH)
        b1 = flat[NS * NCK0 * CH:].reshape(NS, NCK1, CH)
        b0 = jnp.pad(b0, ((0, 0), (0, NCK_MAX - NCK0), (0, 0)),
                     constant_values=N_PAD - 1)
        b1 = jnp.pad(b1, ((0, 0), (0, NCK_MAX - NCK1), (0, 0)),
                     constant_values=N_PAD - 1)
        return jnp.concatenate([b0, b1], axis=0)

    src3 = _pack(src)
    dst3 = _pack(dst)
    batch_p = jnp.full((1, N_PAD), G, jnp.int32).at[0, :N].set(
        batch.astype(jnp.int32))

    cnt_parts = _sc_degree(dst3)
    cnt2 = cnt_parts[:, :, 0].T

    a, b = _tc_pre(x_pad, Wl0.T, Wr0.T)
    biases_next = [(bl0, Wl1, Wr1), (bl1, Wl2, Wr2)]
    for bl, Wl_n, Wr_n in biases_next:
        parts = _sc_scatter_add(a, src3, dst3)
        a, b = _tc_mid(parts, cnt2, b, bl.reshape(1, D), Wl_n.T, Wr_n.T)
    parts = _sc_scatter_add(a, src3, dst3)
    return _tc_final(parts, cnt2, b, bl2.reshape(1, D), batch_p)


# speedup vs baseline: 3.6939x; 3.6939x over previous
import functools

import jax
import jax.numpy as jnp
from jax import lax
from jax.experimental import pallas as pl
from jax.experimental.pallas import tpu as pltpu
from jax.experimental.pallas import tpu_sc as plsc

N = 10000
E = 320000
D = 128
G = 64

NC = 2
NS = 16
NW = NC * NS
CH = 128
NCHUNK = 80
EPT = NCHUNK * CH
E_PAD = EPT * NW
N_PAD = 10240
RPT = N_PAD // NS

_F32 = jnp.float32
_MESH = plsc.VectorSubcoreMesh(core_axis_name="c", subcore_axis_name="s")


WIN = 8
NWIN = NCHUNK // WIN
NCK0 = 80
NCK1 = 80
NCK_MAX = 80


def _zero_fill(buf, val):
    v16 = jnp.full((16,), val, _F32)

    def _row(r, carry):
        for q in range(D // 16):
            buf[r, pl.ds(q * 16, 16)] = v16
        return carry

    lax.fori_loop(0, CH, _row, 0)


def _zero_agg(buf, agg_sh, s):
    _zero_fill(buf, 0.0)
    r0 = s * RPT
    for k in range(RPT // CH):
        pltpu.sync_copy(buf, agg_sh.at[pl.ds(r0 + k * CH, CH)])


def _writeout(buf, agg_sh, out_hbm, c, s):
    r0 = s * RPT
    for k in range(RPT // CH):
        pltpu.sync_copy(agg_sh.at[pl.ds(r0 + k * CH, CH)], buf)
        pltpu.sync_copy(buf, out_hbm.at[c, pl.ds(r0 + k * CH, CH)])


@functools.partial(
    pl.kernel,
    out_type=jax.ShapeDtypeStruct((NC, N_PAD, D), _F32),
    mesh=_MESH,
    scratch_types=[
        pltpu.VMEM((WIN, CH), jnp.int32),
        pltpu.VMEM((WIN, CH), jnp.int32),
        pltpu.VMEM((CH, D), _F32),
        pltpu.VMEM((CH, D), _F32),
        pltpu.VMEM_SHARED((N_PAD, D), _F32),
        pltpu.SemaphoreType.DMA,
        pltpu.SemaphoreType.DMA,
    ],
)
def _sc_scatter_add(a_hbm, src_hbm, dst_hbm, out_hbm,
                    src_w, dst_w, buf0, buf1, agg_sh, sem0, sem1):
    c = lax.axis_index("c")
    s = lax.axis_index("s")
    wid = c * NS + s
    nwin = lax.select(c == 0, NCK0 // WIN, NCK1 // WIN)

    _zero_agg(buf0, agg_sh, s)
    plsc.subcore_barrier()

    bufs = (buf0, buf1)
    sems = (sem0, sem1)

    def _window(w, carry):
        pltpu.sync_copy(src_hbm.at[wid, pl.ds(w * WIN, WIN)], src_w)
        pltpu.sync_copy(dst_hbm.at[wid, pl.ds(w * WIN, WIN)], dst_w)
        pending = pltpu.async_copy(a_hbm.at[src_w.at[0]], buf0, sem0)
        for k in range(WIN):
            nxt = None
            if k + 1 < WIN:
                nxt = pltpu.async_copy(
                    a_hbm.at[src_w.at[k + 1]], bufs[(k + 1) % 2],
                    sems[(k + 1) % 2])
            pending.wait()
            pltpu.sync_copy(bufs[k % 2], agg_sh.at[dst_w.at[k]], add=True)
            pending = nxt
        return carry

    lax.fori_loop(0, nwin, _window, 0)
    plsc.subcore_barrier()
    _writeout(buf0, agg_sh, out_hbm, c, s)


@functools.partial(
    pl.kernel,
    out_type=jax.ShapeDtypeStruct((NC, N_PAD, D), _F32),
    mesh=_MESH,
    scratch_types=[
        pltpu.VMEM((WIN, CH), jnp.int32),
        pltpu.VMEM((CH, D), _F32),
        pltpu.VMEM_SHARED((N_PAD, D), _F32),
    ],
)
def _sc_degree(dst_hbm, out_hbm, dst_w, buf, agg_sh):
    c = lax.axis_index("c")
    s = lax.axis_index("s")
    wid = c * NS + s
    nwin = lax.select(c == 0, NCK0 // WIN, NCK1 // WIN)

    _zero_agg(buf, agg_sh, s)
    plsc.subcore_barrier()
    _zero_fill(buf, 1.0)

    def _window(w, carry):
        pltpu.sync_copy(dst_hbm.at[wid, pl.ds(w * WIN, WIN)], dst_w)
        for k in range(WIN):
            pltpu.sync_copy(buf, agg_sh.at[dst_w.at[k]], add=True)
        return carry

    lax.fori_loop(0, nwin, _window, 0)
    plsc.subcore_barrier()
    _writeout(buf, agg_sh, out_hbm, c, s)


BR = 2560
NBLK = N_PAD // BR


def _dot(a, b):
    return jnp.dot(a, b, precision=lax.Precision.HIGHEST,
                   preferred_element_type=_F32)


def _pre_body(x_ref, wlT_ref, wrT_ref, a_ref, b_ref):
    xv = x_ref[...]
    a_ref[...] = _dot(xv, wlT_ref[...])
    b_ref[...] = _dot(xv, wrT_ref[...])


_W_SPEC = pl.BlockSpec((D, D), lambda i: (0, 0))
_ROW_SPEC = pl.BlockSpec((BR, D), lambda i: (i, 0))

_tc_pre = pl.pallas_call(
    _pre_body,
    grid=(NBLK,),
    in_specs=[_ROW_SPEC, _W_SPEC, _W_SPEC],
    out_specs=(_ROW_SPEC, _ROW_SPEC),
    out_shape=(jax.ShapeDtypeStruct((N_PAD, D), _F32),
               jax.ShapeDtypeStruct((N_PAD, D), _F32)),
)


def _relu_layer(p_ref, cnt_ref, b_ref, bias_ref):
    agg = p_ref[0] + p_ref[1]
    cnt = cnt_ref[:, 0:1] + cnt_ref[:, 1:2]
    inv = 1.0 / jnp.maximum(cnt, 1.0)
    h = jnp.maximum(agg * inv + bias_ref[...] + b_ref[...], 0.0)
    row0 = pl.program_id(0) * BR
    mask = (row0 + lax.broadcasted_iota(jnp.int32, (BR, 1), 0) < N)
    return h * mask.astype(_F32)


_P_SPEC = pl.BlockSpec((NC, BR, D), lambda i: (0, i, 0))
_CNT_SPEC = pl.BlockSpec((BR, NC), lambda i: (i, 0))
_BIAS_SPEC = pl.BlockSpec((1, D), lambda i: (0, 0))


def _mid_body(p_ref, cnt_ref, b_ref, bias_ref, wlT_ref, wrT_ref,
              a_out, b_out):
    h = _relu_layer(p_ref, cnt_ref, b_ref, bias_ref)
    a_out[...] = _dot(h, wlT_ref[...])
    b_out[...] = _dot(h, wrT_ref[...])


_tc_mid = pl.pallas_call(
    _mid_body,
    grid=(NBLK,),
    in_specs=[_P_SPEC, _CNT_SPEC, _ROW_SPEC, _BIAS_SPEC, _W_SPEC, _W_SPEC],
    out_specs=(_ROW_SPEC, _ROW_SPEC),
    out_shape=(jax.ShapeDtypeStruct((N_PAD, D), _F32),
               jax.ShapeDtypeStruct((N_PAD, D), _F32)),
)


def _final_body(p_ref, cnt_ref, b_ref, bias_ref, batch_ref, out_ref):
    h = _relu_layer(p_ref, cnt_ref, b_ref, bias_ref)
    onehot = (lax.broadcasted_iota(jnp.int32, (G, BR), 0)
              == batch_ref[...]).astype(_F32)
    part = _dot(onehot, h)

    @pl.when(pl.program_id(0) == 0)
    def _init():
        out_ref[...] = jnp.zeros((G, D), _F32)

    out_ref[...] += part


_tc_final = pl.pallas_call(
    _final_body,
    grid=(NBLK,),
    in_specs=[_P_SPEC, _CNT_SPEC, _ROW_SPEC, _BIAS_SPEC,
              pl.BlockSpec((1, BR), lambda i: (0, i))],
    out_specs=pl.BlockSpec((G, D), lambda i: (0, 0)),
    out_shape=jax.ShapeDtypeStruct((G, D), _F32),
)


def kernel(x, edge_index, batch, Wl0, bl0, Wr0, Wl1, bl1, Wr1, Wl2, bl2, Wr2):
    src = edge_index[0].astype(jnp.int32)
    dst = edge_index[1].astype(jnp.int32)

    x_pad = jnp.zeros((N_PAD, D), _F32).at[:N].set(x)
    def _pack(e, spread_pad):
        if spread_pad:
            pad = (jnp.arange(E_PAD, dtype=jnp.int32) * 37) % N
            flat = pad.at[:E].set(e)
        else:
            flat = jnp.full((E_PAD,), N_PAD - 1, jnp.int32).at[:E].set(e)
        b0 = flat[:NS * NCK0 * CH].reshape(NS, NCK0, CH)
        b1 = flat[NS * NCK0 * CH:].reshape(NS, NCK1, CH)
        b0 = jnp.pad(b0, ((0, 0), (0, NCK_MAX - NCK0), (0, 0)),
                     constant_values=N_PAD - 1)
        b1 = jnp.pad(b1, ((0, 0), (0, NCK_MAX - NCK1), (0, 0)),
                     constant_values=N_PAD - 1)
        return jnp.concatenate([b0, b1], axis=0)

    src3 = _pack(src, True)
    dst3 = _pack(dst, False)
    batch_p = jnp.full((1, N_PAD), G, jnp.int32).at[0, :N].set(
        batch.astype(jnp.int32))

    cnt_parts = _sc_degree(dst3)
    cnt2 = cnt_parts[:, :, 0].T

    a, b = _tc_pre(x_pad, Wl0.T, Wr0.T)
    biases_next = [(bl0, Wl1, Wr1), (bl1, Wl2, Wr2)]
    for bl, Wl_n, Wr_n in biases_next:
        parts = _sc_scatter_add(a, src3, dst3)
        a, b = _tc_mid(parts, cnt2, b, bl.reshape(1, D), Wl_n.T, Wr_n.T)
    parts = _sc_scatter_add(a, src3, dst3)
    return _tc_final(parts, cnt2, b, bl2.reshape(1, D), batch_p)


# speedup vs baseline: 3.7687x; 1.0202x over previous
import functools

import jax
import jax.numpy as jnp
from jax import lax
from jax.experimental import pallas as pl
from jax.experimental.pallas import tpu as pltpu
from jax.experimental.pallas import tpu_sc as plsc

N = 10000
E = 320000
D = 128
G = 64

NC = 2
NS = 16
NW = NC * NS
CH = 128
NCHUNK = 80
EPT = NCHUNK * CH
E_PAD = EPT * NW
N_PAD = 10240
RPT = N_PAD // NS

_F32 = jnp.float32
_MESH = plsc.VectorSubcoreMesh(core_axis_name="c", subcore_axis_name="s")


WIN = 8
NWIN = NCHUNK // WIN
NCK0 = 80
NCK1 = 80
NCK_MAX = 80


def _zero_fill(buf, val):
    v16 = jnp.full((16,), val, _F32)

    def _row(r, carry):
        for q in range(D // 16):
            buf[r, pl.ds(q * 16, 16)] = v16
        return carry

    lax.fori_loop(0, CH, _row, 0)


def _zero_agg(buf, agg_sh, s):
    _zero_fill(buf, 0.0)
    r0 = s * RPT
    for k in range(RPT // CH):
        pltpu.sync_copy(buf, agg_sh.at[pl.ds(r0 + k * CH, CH)])


def _writeout(buf, agg_sh, out_hbm, c, s):
    r0 = s * RPT
    for k in range(RPT // CH):
        pltpu.sync_copy(agg_sh.at[pl.ds(r0 + k * CH, CH)], buf)
        pltpu.sync_copy(buf, out_hbm.at[c, pl.ds(r0 + k * CH, CH)])


@functools.partial(
    pl.kernel,
    out_type=jax.ShapeDtypeStruct((NC, N_PAD, D), _F32),
    mesh=_MESH,
    scratch_types=[
        pltpu.VMEM((WIN, CH), jnp.int32),
        pltpu.VMEM((WIN, CH), jnp.int32),
        pltpu.VMEM((CH, D), _F32),
        pltpu.VMEM((CH, D), _F32),
        pltpu.VMEM_SHARED((N_PAD, D), _F32),
        pltpu.SemaphoreType.DMA,
        pltpu.SemaphoreType.DMA,
    ],
)
def _sc_scatter_add(a_hbm, src_hbm, dst_hbm, out_hbm,
                    src_w, dst_w, buf0, buf1, agg_sh, sem0, sem1):
    c = lax.axis_index("c")
    s = lax.axis_index("s")
    wid = c * NS + s
    nwin = lax.select(c == 0, NCK0 // WIN, NCK1 // WIN)

    _zero_agg(buf0, agg_sh, s)
    plsc.subcore_barrier()

    bufs = (buf0, buf1)
    sems = (sem0, sem1)

    def _window(w, carry):
        pltpu.sync_copy(src_hbm.at[wid, pl.ds(w * WIN, WIN)], src_w)
        pltpu.sync_copy(dst_hbm.at[wid, pl.ds(w * WIN, WIN)], dst_w)
        pending = pltpu.async_copy(a_hbm.at[src_w.at[0]], buf0, sem0)
        for k in range(WIN):
            nxt = None
            if k + 1 < WIN:
                nxt = pltpu.async_copy(
                    a_hbm.at[src_w.at[k + 1]], bufs[(k + 1) % 2],
                    sems[(k + 1) % 2])
            pending.wait()
            pltpu.sync_copy(bufs[k % 2], agg_sh.at[dst_w.at[k]], add=True)
            pending = nxt
        return carry

    lax.fori_loop(0, nwin, _window, 0)
    plsc.subcore_barrier()
    _writeout(buf0, agg_sh, out_hbm, c, s)


@functools.partial(
    pl.kernel,
    out_type=jax.ShapeDtypeStruct((NC, N_PAD, D), _F32),
    mesh=_MESH,
    scratch_types=[
        pltpu.VMEM((WIN, CH), jnp.int32),
        pltpu.VMEM((CH, D), _F32),
        pltpu.VMEM_SHARED((N_PAD, D), _F32),
    ],
)
def _sc_degree(dst_hbm, out_hbm, dst_w, buf, agg_sh):
    c = lax.axis_index("c")
    s = lax.axis_index("s")
    wid = c * NS + s
    nwin = lax.select(c == 0, NCK0 // WIN, NCK1 // WIN)

    _zero_agg(buf, agg_sh, s)
    plsc.subcore_barrier()
    _zero_fill(buf, 1.0)

    def _window(w, carry):
        pltpu.sync_copy(dst_hbm.at[wid, pl.ds(w * WIN, WIN)], dst_w)
        for k in range(WIN):
            pltpu.sync_copy(buf, agg_sh.at[dst_w.at[k]], add=True)
        return carry

    lax.fori_loop(0, nwin, _window, 0)
    plsc.subcore_barrier()
    _writeout(buf, agg_sh, out_hbm, c, s)


BR = 2560
NBLK = N_PAD // BR


def _dot(a, b):
    return jnp.dot(a, b, preferred_element_type=_F32)


def _pre_body(x_ref, wlT_ref, wrT_ref, a_ref, b_ref):
    xv = x_ref[...]
    a_ref[...] = _dot(xv, wlT_ref[...])
    b_ref[...] = _dot(xv, wrT_ref[...])


_W_SPEC = pl.BlockSpec((D, D), lambda i: (0, 0))
_ROW_SPEC = pl.BlockSpec((BR, D), lambda i: (i, 0))

_tc_pre = pl.pallas_call(
    _pre_body,
    grid=(NBLK,),
    in_specs=[_ROW_SPEC, _W_SPEC, _W_SPEC],
    out_specs=(_ROW_SPEC, _ROW_SPEC),
    out_shape=(jax.ShapeDtypeStruct((N_PAD, D), _F32),
               jax.ShapeDtypeStruct((N_PAD, D), _F32)),
)


def _relu_layer(p_ref, cnt_ref, b_ref, bias_ref):
    agg = p_ref[0] + p_ref[1]
    cnt = cnt_ref[:, 0:1] + cnt_ref[:, 1:2]
    inv = 1.0 / jnp.maximum(cnt, 1.0)
    h = jnp.maximum(agg * inv + bias_ref[...] + b_ref[...], 0.0)
    row0 = pl.program_id(0) * BR
    mask = (row0 + lax.broadcasted_iota(jnp.int32, (BR, 1), 0) < N)
    return h * mask.astype(_F32)


_P_SPEC = pl.BlockSpec((NC, BR, D), lambda i: (0, i, 0))
_CNT_SPEC = pl.BlockSpec((BR, NC), lambda i: (i, 0))
_BIAS_SPEC = pl.BlockSpec((1, D), lambda i: (0, 0))


def _mid_body(p_ref, cnt_ref, b_ref, bias_ref, wlT_ref, wrT_ref,
              a_out, b_out):
    h = _relu_layer(p_ref, cnt_ref, b_ref, bias_ref)
    a_out[...] = _dot(h, wlT_ref[...])
    b_out[...] = _dot(h, wrT_ref[...])


_tc_mid = pl.pallas_call(
    _mid_body,
    grid=(NBLK,),
    in_specs=[_P_SPEC, _CNT_SPEC, _ROW_SPEC, _BIAS_SPEC, _W_SPEC, _W_SPEC],
    out_specs=(_ROW_SPEC, _ROW_SPEC),
    out_shape=(jax.ShapeDtypeStruct((N_PAD, D), _F32),
               jax.ShapeDtypeStruct((N_PAD, D), _F32)),
)


def _final_body(p_ref, cnt_ref, b_ref, bias_ref, batch_ref, out_ref):
    h = _relu_layer(p_ref, cnt_ref, b_ref, bias_ref)
    onehot = (lax.broadcasted_iota(jnp.int32, (G, BR), 0)
              == batch_ref[...]).astype(_F32)
    part = _dot(onehot, h)

    @pl.when(pl.program_id(0) == 0)
    def _init():
        out_ref[...] = jnp.zeros((G, D), _F32)

    out_ref[...] += part


_tc_final = pl.pallas_call(
    _final_body,
    grid=(NBLK,),
    in_specs=[_P_SPEC, _CNT_SPEC, _ROW_SPEC, _BIAS_SPEC,
              pl.BlockSpec((1, BR), lambda i: (0, i))],
    out_specs=pl.BlockSpec((G, D), lambda i: (0, 0)),
    out_shape=jax.ShapeDtypeStruct((G, D), _F32),
)


def kernel(x, edge_index, batch, Wl0, bl0, Wr0, Wl1, bl1, Wr1, Wl2, bl2, Wr2):
    src = edge_index[0].astype(jnp.int32)
    dst = edge_index[1].astype(jnp.int32)

    x_pad = jnp.zeros((N_PAD, D), _F32).at[:N].set(x)
    def _pack(e, spread_pad):
        if spread_pad:
            pad = (jnp.arange(E_PAD, dtype=jnp.int32) * 37) % N
            flat = pad.at[:E].set(e)
        else:
            flat = jnp.full((E_PAD,), N_PAD - 1, jnp.int32).at[:E].set(e)
        b0 = flat[:NS * NCK0 * CH].reshape(NS, NCK0, CH)
        b1 = flat[NS * NCK0 * CH:].reshape(NS, NCK1, CH)
        b0 = jnp.pad(b0, ((0, 0), (0, NCK_MAX - NCK0), (0, 0)),
                     constant_values=N_PAD - 1)
        b1 = jnp.pad(b1, ((0, 0), (0, NCK_MAX - NCK1), (0, 0)),
                     constant_values=N_PAD - 1)
        return jnp.concatenate([b0, b1], axis=0)

    src3 = _pack(src, True)
    dst3 = _pack(dst, False)
    batch_p = jnp.full((1, N_PAD), G, jnp.int32).at[0, :N].set(
        batch.astype(jnp.int32))

    cnt_parts = _sc_degree(dst3)
    cnt2 = cnt_parts[:, :, 0].T

    a, b = _tc_pre(x_pad, Wl0.T, Wr0.T)
    biases_next = [(bl0, Wl1, Wr1), (bl1, Wl2, Wr2)]
    for bl, Wl_n, Wr_n in biases_next:
        parts = _sc_scatter_add(a, src3, dst3)
        a, b = _tc_mid(parts, cnt2, b, bl.reshape(1, D), Wl_n.T, Wr_n.T)
    parts = _sc_scatter_add(a, src3, dst3)
    return _tc_final(parts, cnt2, b, bl2.reshape(1, D), batch_p)


# speedup vs baseline: 3.9370x; 1.0447x over previous
import functools

import jax
import jax.numpy as jnp
from jax import lax
from jax.experimental import pallas as pl
from jax.experimental.pallas import tpu as pltpu
from jax.experimental.pallas import tpu_sc as plsc

N = 10000
E = 320000
D = 128
G = 64

NC = 2
NS = 16
NW = NC * NS
CH = 112
NCHUNK = 96
EPT = NCHUNK * CH
E_PAD = EPT * NW
N_PAD = 10240
RPT = N_PAD // NS
NWIN = 12

_F32 = jnp.float32
_MESH = plsc.VectorSubcoreMesh(core_axis_name="c", subcore_axis_name="s")


WIN = 8
NBUF = 3
_RW = [(0, 112), (112, 112), (224, 112), (336, 112), (448, 112), (560, 80)]


def _zero_fill(buf, val):
    v16 = jnp.full((16,), val, _F32)

    def _row(r, carry):
        for q in range(D // 16):
            buf[r, pl.ds(q * 16, 16)] = v16
        return carry

    lax.fori_loop(0, CH, _row, 0)


def _zero_agg(buf, agg_sh, s):
    _zero_fill(buf, 0.0)
    r0 = s * RPT
    for off, sz in _RW:
        pltpu.sync_copy(buf.at[pl.ds(0, sz)], agg_sh.at[pl.ds(r0 + off, sz)])


def _writeout(buf, agg_sh, out_hbm, c, s):
    r0 = s * RPT
    for off, sz in _RW:
        pltpu.sync_copy(agg_sh.at[pl.ds(r0 + off, sz)], buf.at[pl.ds(0, sz)])
        pltpu.sync_copy(buf.at[pl.ds(0, sz)],
                        out_hbm.at[c, pl.ds(r0 + off, sz)])


@functools.partial(
    pl.kernel,
    out_type=jax.ShapeDtypeStruct((NC, N_PAD, D), _F32),
    mesh=_MESH,
    scratch_types=[
        pltpu.VMEM((2, WIN, CH), jnp.int32),
        pltpu.VMEM((2, WIN, CH), jnp.int32),
        pltpu.VMEM((NBUF, CH, D), _F32),
        pltpu.VMEM_SHARED((N_PAD, D), _F32),
        pltpu.SemaphoreType.DMA,
        pltpu.SemaphoreType.DMA,
        pltpu.SemaphoreType.DMA,
        pltpu.SemaphoreType.DMA,
        pltpu.SemaphoreType.DMA,
        pltpu.SemaphoreType.DMA,
        pltpu.SemaphoreType.DMA,
        pltpu.SemaphoreType.DMA,
    ],
)
def _sc_scatter_add(a_hbm, src_hbm, dst_hbm, out_hbm,
                    src_w, dst_w, bufs, agg_sh,
                    gs0, gs1, gs2, ss0, ss1, ss2, semi0, semi1):
    c = lax.axis_index("c")
    s = lax.axis_index("s")
    wid = c * NS + s
    gs = (gs0, gs1, gs2)
    ss = (ss0, ss1, ss2)

    pltpu.sync_copy(src_hbm.at[wid, pl.ds(0, WIN)], src_w.at[0])
    pltpu.sync_copy(dst_hbm.at[wid, pl.ds(0, WIN)], dst_w.at[0])
    _zero_agg(bufs.at[0], agg_sh, s)
    plsc.subcore_barrier()

    def _window(w, carry):
        p = lax.rem(w, 2)
        pn = lax.rem(w + 1, 2)
        wn = lax.rem(w + 1, NWIN)
        hi0 = pltpu.async_copy(
            src_hbm.at[wid, pl.ds(wn * WIN, WIN)], src_w.at[pn], semi0)
        hi1 = pltpu.async_copy(
            dst_hbm.at[wid, pl.ds(wn * WIN, WIN)], dst_w.at[pn], semi1)
        g, sc = {}, {}
        for k in range(NBUF - 1):
            g[k] = pltpu.async_copy(
                a_hbm.at[src_w.at[p, k]], bufs.at[k % NBUF], gs[k % NBUF])
        for k in range(WIN):
            kk = k + NBUF - 1
            if kk < WIN:
                if k - 1 >= 0:
                    sc[k - 1].wait()
                g[kk] = pltpu.async_copy(
                    a_hbm.at[src_w.at[p, kk]], bufs.at[kk % NBUF],
                    gs[kk % NBUF])
            g[k].wait()
            sc[k] = pltpu.async_copy(
                bufs.at[k % NBUF], agg_sh.at[dst_w.at[p, k]], ss[k % NBUF],
                add=True)
        for k in range(WIN - NBUF, WIN):
            if k >= 0:
                sc[k].wait()
        hi0.wait()
        hi1.wait()
        return carry

    lax.fori_loop(0, NWIN, _window, 0)
    plsc.subcore_barrier()
    _writeout(bufs.at[0], agg_sh, out_hbm, c, s)


@functools.partial(
    pl.kernel,
    out_type=jax.ShapeDtypeStruct((NC, N_PAD, D), _F32),
    mesh=_MESH,
    scratch_types=[
        pltpu.VMEM((WIN, CH), jnp.int32),
        pltpu.VMEM((CH, D), _F32),
        pltpu.VMEM_SHARED((N_PAD, D), _F32),
    ],
)
def _sc_degree(dst_hbm, out_hbm, dst_w, buf, agg_sh):
    c = lax.axis_index("c")
    s = lax.axis_index("s")
    wid = c * NS + s

    _zero_agg(buf, agg_sh, s)
    plsc.subcore_barrier()
    _zero_fill(buf, 1.0)

    def _window(w, carry):
        pltpu.sync_copy(dst_hbm.at[wid, pl.ds(w * WIN, WIN)], dst_w)
        for k in range(WIN):
            pltpu.sync_copy(buf, agg_sh.at[dst_w.at[k]], add=True)
        return carry

    lax.fori_loop(0, NWIN, _window, 0)
    plsc.subcore_barrier()
    _writeout(buf, agg_sh, out_hbm, c, s)


BR = 2560
NBLK = N_PAD // BR


def _dot(a, b):
    return jnp.dot(a, b, preferred_element_type=_F32)


def _pre_body(x_ref, wlT_ref, wrT_ref, a_ref, b_ref):
    xv = x_ref[...]
    a_ref[...] = _dot(xv, wlT_ref[...])
    b_ref[...] = _dot(xv, wrT_ref[...])


_W_SPEC = pl.BlockSpec((D, D), lambda i: (0, 0))
_ROW_SPEC = pl.BlockSpec((BR, D), lambda i: (i, 0))

_tc_pre = pl.pallas_call(
    _pre_body,
    grid=(NBLK,),
    in_specs=[_ROW_SPEC, _W_SPEC, _W_SPEC],
    out_specs=(_ROW_SPEC, _ROW_SPEC),
    out_shape=(jax.ShapeDtypeStruct((N_PAD, D), _F32),
               jax.ShapeDtypeStruct((N_PAD, D), _F32)),
)


def _relu_layer(p_ref, cnt_ref, b_ref, bias_ref):
    agg = p_ref[0] + p_ref[1]
    cnt = cnt_ref[:, 0:1] + cnt_ref[:, 1:2]
    inv = 1.0 / jnp.maximum(cnt, 1.0)
    h = jnp.maximum(agg * inv + bias_ref[...] + b_ref[...], 0.0)
    row0 = pl.program_id(0) * BR
    mask = (row0 + lax.broadcasted_iota(jnp.int32, (BR, 1), 0) < N)
    return h * mask.astype(_F32)


_P_SPEC = pl.BlockSpec((NC, BR, D), lambda i: (0, i, 0))
_CNT_SPEC = pl.BlockSpec((BR, NC), lambda i: (i, 0))
_BIAS_SPEC = pl.BlockSpec((1, D), lambda i: (0, 0))


def _mid_body(p_ref, cnt_ref, b_ref, bias_ref, wlT_ref, wrT_ref,
              a_out, b_out):
    h = _relu_layer(p_ref, cnt_ref, b_ref, bias_ref)
    a_out[...] = _dot(h, wlT_ref[...])
    b_out[...] = _dot(h, wrT_ref[...])


_tc_mid = pl.pallas_call(
    _mid_body,
    grid=(NBLK,),
    in_specs=[_P_SPEC, _CNT_SPEC, _ROW_SPEC, _BIAS_SPEC, _W_SPEC, _W_SPEC],
    out_specs=(_ROW_SPEC, _ROW_SPEC),
    out_shape=(jax.ShapeDtypeStruct((N_PAD, D), _F32),
               jax.ShapeDtypeStruct((N_PAD, D), _F32)),
)


def _final_body(p_ref, cnt_ref, b_ref, bias_ref, batch_ref, out_ref):
    h = _relu_layer(p_ref, cnt_ref, b_ref, bias_ref)
    onehot = (lax.broadcasted_iota(jnp.int32, (G, BR), 0)
              == batch_ref[...]).astype(_F32)
    part = _dot(onehot, h)

    @pl.when(pl.program_id(0) == 0)
    def _init():
        out_ref[...] = jnp.zeros((G, D), _F32)

    out_ref[...] += part


_tc_final = pl.pallas_call(
    _final_body,
    grid=(NBLK,),
    in_specs=[_P_SPEC, _CNT_SPEC, _ROW_SPEC, _BIAS_SPEC,
              pl.BlockSpec((1, BR), lambda i: (0, i))],
    out_specs=pl.BlockSpec((G, D), lambda i: (0, 0)),
    out_shape=jax.ShapeDtypeStruct((G, D), _F32),
)


def kernel(x, edge_index, batch, Wl0, bl0, Wr0, Wl1, bl1, Wr1, Wl2, bl2, Wr2):
    src = edge_index[0].astype(jnp.int32)
    dst = edge_index[1].astype(jnp.int32)

    x_pad = jnp.zeros((N_PAD, D), _F32).at[:N].set(x)
    def _pack(e, spread_pad):
        if spread_pad:
            pad = (jnp.arange(E_PAD, dtype=jnp.int32) * 37) % N
            flat = pad.at[:E].set(e)
        else:
            flat = jnp.full((E_PAD,), N_PAD - 1, jnp.int32).at[:E].set(e)
        return flat.reshape(NW, NCHUNK, CH)

    src3 = _pack(src, True)
    dst3 = _pack(dst, False)
    batch_p = jnp.full((1, N_PAD), G, jnp.int32).at[0, :N].set(
        batch.astype(jnp.int32))

    cnt_parts = _sc_degree(dst3)
    cnt2 = cnt_parts[:, :, 0].T

    a, b = _tc_pre(x_pad, Wl0.T, Wr0.T)
    biases_next = [(bl0, Wl1, Wr1), (bl1, Wl2, Wr2)]
    for bl, Wl_n, Wr_n in biases_next:
        parts = _sc_scatter_add(a, src3, dst3)
        a, b = _tc_mid(parts, cnt2, b, bl.reshape(1, D), Wl_n.T, Wr_n.T)
    parts = _sc_scatter_add(a, src3, dst3)
    return _tc_final(parts, cnt2, b, bl2.reshape(1, D), batch_p)
